# speedup vs baseline: 7.4178x; 7.4178x over previous
import functools

import jax
import jax.numpy as jnp
from jax import lax
from jax.experimental import pallas as pl
from jax.experimental.pallas import tpu as pltpu
from jax.experimental.pallas import tpu_sc as plsc

_N = 10000
_E = 320000
_D = 128

_NC = 2
_NS = 16
_NW = _NC * _NS
_EPW = _E // _NW
_B = 80
_K = _EPW // _B
_ROWS_PER_TILE = 640


def _sc_segment_sum(x, src_r, dst_r):
    mesh = plsc.VectorSubcoreMesh(core_axis_name="c", subcore_axis_name="s")

    @functools.partial(
        pl.kernel,
        out_type=jax.ShapeDtypeStruct((_NC, _N, _D), jnp.float32),
        mesh=mesh,
        scratch_types=[
            pltpu.VMEM((_K, _B), jnp.int32),
            pltpu.VMEM((_K, _B), jnp.int32),
            pltpu.VMEM((_B, _D), jnp.float32),
            pltpu.VMEM_SHARED((_N, _D), jnp.float32),
            pltpu.SemaphoreType.DMA,
        ],
    )
    def k(x_hbm, src_hbm, dst_hbm, out_hbm, src_v, dst_v, rows_v, acc_sh, sem):
        c = lax.axis_index("c")
        s = lax.axis_index("s")
        wid = s * _NC + c

        def zero_body(i, _):
            zr = i // (_D // 16)
            zc = (i % (_D // 16)) * 16
            rows_v[zr, pl.ds(zc, 16)] = jnp.zeros((16,), jnp.float32)
            return 0

        lax.fori_loop(0, _B * (_D // 16), zero_body, 0)

        zbase = jnp.minimum(s * _ROWS_PER_TILE, _N - _ROWS_PER_TILE)
        for rblk in range(_ROWS_PER_TILE // _B):
            pltpu.sync_copy(rows_v, acc_sh.at[pl.ds(zbase + rblk * _B, _B)])

        pltpu.sync_copy(src_hbm.at[wid], src_v)
        pltpu.sync_copy(dst_hbm.at[wid], dst_v)

        plsc.subcore_barrier()

        def body(j, _):
            pltpu.async_copy(x_hbm.at[src_v.at[j]], rows_v, sem).wait()
            pltpu.sync_copy(rows_v, acc_sh.at[dst_v.at[j]], add=True)
            return 0

        lax.fori_loop(0, _K, body, 0)

        plsc.subcore_barrier()

        @pl.when(s < _NS - 1)
        def _():
            pltpu.sync_copy(
                acc_sh.at[pl.ds(s * _ROWS_PER_TILE, _ROWS_PER_TILE)],
                out_hbm.at[c, pl.ds(s * _ROWS_PER_TILE, _ROWS_PER_TILE)],
            )

        @pl.when(s == _NS - 1)
        def _():
            last = (_NS - 1) * _ROWS_PER_TILE
            pltpu.sync_copy(
                acc_sh.at[pl.ds(last, _N - last)],
                out_hbm.at[c, pl.ds(last, _N - last)],
            )

    return k(x, src_r, dst_r)


_TR = 2000


def _tc_linear(x, p, w, b, lsm):
    def body(x_ref, p_ref, w_ref, b_ref, o_ref):
        rst = x_ref[...] + p_ref[0] + p_ref[1]
        h = jnp.dot(rst, w_ref[...], preferred_element_type=jnp.float32) + b_ref[...]
        if lsm:
            m = jnp.max(h, axis=-1, keepdims=True)
            e = jnp.exp(h - m)
            h = h - m - jnp.log(jnp.sum(e, axis=-1, keepdims=True))
        o_ref[...] = h

    return pl.pallas_call(
        body,
        grid=(_N // _TR,),
        in_specs=[
            pl.BlockSpec((_TR, _D), lambda i: (i, 0)),
            pl.BlockSpec((_NC, _TR, _D), lambda i: (0, i, 0)),
            pl.BlockSpec((_D, _D), lambda i: (0, 0)),
            pl.BlockSpec((1, _D), lambda i: (0, 0)),
        ],
        out_specs=pl.BlockSpec((_TR, _D), lambda i: (i, 0)),
        out_shape=jax.ShapeDtypeStruct((_N, _D), jnp.float32),
    )(x, p, w, b)


def kernel(features, edge_index, W1, b1, W2, b2):
    src_r = edge_index[0].reshape(_NW, _K, _B)
    dst_r = edge_index[1].reshape(_NW, _K, _B)
    b1r = b1.reshape(1, _D)
    b2r = b2.reshape(1, _D)

    p1 = _sc_segment_sum(features, src_r, dst_r)
    h1 = _tc_linear(features, p1, W1, b1r, lsm=False)
    p2 = _sc_segment_sum(h1, src_r, dst_r)
    return _tc_linear(h1, p2, W2, b2r, lsm=True)


# speedup vs baseline: 11.9669x; 1.6133x over previous
import functools

import jax
import jax.numpy as jnp
from jax import lax
from jax.experimental import pallas as pl
from jax.experimental.pallas import tpu as pltpu
from jax.experimental.pallas import tpu_sc as plsc

_N = 10000
_E = 320000
_D = 128

_NC = 2
_NS = 16
_NW = _NC * _NS
_EPW = _E // _NW
_B = 80
_K = _EPW // _B
_ROWS_PER_TILE = 640


def _sc_segment_sum(x, combo_r):
    mesh = plsc.VectorSubcoreMesh(core_axis_name="c", subcore_axis_name="s")

    @functools.partial(
        pl.kernel,
        out_type=jax.ShapeDtypeStruct((_NC, _N, _D), jnp.float32),
        mesh=mesh,
        scratch_types=[
            pltpu.VMEM((_K, _B), jnp.int32),
            pltpu.VMEM((_B,), jnp.int32),
            pltpu.VMEM((_B,), jnp.int32),
            pltpu.VMEM((_B,), jnp.int32),
            pltpu.VMEM((_B,), jnp.int32),
            pltpu.VMEM((_B, _D), jnp.float32),
            pltpu.VMEM((_B, _D), jnp.float32),
            pltpu.VMEM_SHARED((_N, _D), jnp.float32),
            pltpu.SemaphoreType.DMA,
            pltpu.SemaphoreType.DMA,
        ],
    )
    def k(x_hbm, combo_hbm, out_hbm, combo_v, sidx0, sidx1, didx0, didx1,
          rows0, rows1, acc_sh, sem0, sem1):
        c = lax.axis_index("c")
        s = lax.axis_index("s")
        wid = s * _NC + c

        def zero_body(i, _):
            zr = i // (_D // 16)
            zc = (i % (_D // 16)) * 16
            rows0[zr, pl.ds(zc, 16)] = jnp.zeros((16,), jnp.float32)
            return 0

        lax.fori_loop(0, _B * (_D // 16), zero_body, 0)

        zbase = jnp.minimum(s * _ROWS_PER_TILE, _N - _ROWS_PER_TILE)
        for rblk in range(_ROWS_PER_TILE // _B):
            pltpu.sync_copy(rows0, acc_sh.at[pl.ds(zbase + rblk * _B, _B)])

        pltpu.sync_copy(combo_hbm.at[wid], combo_v)

        plsc.subcore_barrier()

        def unpack(j, sidx, didx):
            for i in range(_B // 16):
                w = combo_v[j, pl.ds(i * 16, 16)]
                sidx[pl.ds(i * 16, 16)] = w & 0xFFFF
                didx[pl.ds(i * 16, 16)] = lax.shift_right_logical(w, 16)

        def wait_gather(sidx, buf, sem):
            pltpu.make_async_copy(x_hbm.at[sidx], buf, sem).wait()

        unpack(0, sidx0, didx0)
        pltpu.async_copy(x_hbm.at[sidx0], rows0, sem0)

        def step(cur, csem, cdidx, nxt, nsem, nsidx, ndidx, j):
            unpack(j + 1, nsidx, ndidx)
            pltpu.async_copy(x_hbm.at[nsidx], nxt, nsem)
            wait_gather(nsidx, cur, csem)
            pltpu.sync_copy(cur, acc_sh.at[cdidx], add=True)

        def body(j, _):
            @pl.when(j % 2 == 0)
            def _():
                step(rows0, sem0, didx0, rows1, sem1, sidx1, didx1, j)

            @pl.when(j % 2 == 1)
            def _():
                step(rows1, sem1, didx1, rows0, sem0, sidx0, didx0, j)

            return 0

        lax.fori_loop(0, _K - 1, body, 0)
        wait_gather(sidx0, rows0, sem0)
        pltpu.sync_copy(rows0, acc_sh.at[didx0], add=True)

        plsc.subcore_barrier()

        @pl.when(s < _NS - 1)
        def _():
            pltpu.sync_copy(
                acc_sh.at[pl.ds(s * _ROWS_PER_TILE, _ROWS_PER_TILE)],
                out_hbm.at[c, pl.ds(s * _ROWS_PER_TILE, _ROWS_PER_TILE)],
            )

        @pl.when(s == _NS - 1)
        def _():
            last = (_NS - 1) * _ROWS_PER_TILE
            pltpu.sync_copy(
                acc_sh.at[pl.ds(last, _N - last)],
                out_hbm.at[c, pl.ds(last, _N - last)],
            )

    return k(x, combo_r)


_TR = 2000


def _tc_linear(x, p, w, b, lsm):
    def body(x_ref, p_ref, w_ref, b_ref, o_ref):
        rst = x_ref[...] + p_ref[0] + p_ref[1]
        h = jnp.dot(rst, w_ref[...], preferred_element_type=jnp.float32) + b_ref[...]
        if lsm:
            m = jnp.max(h, axis=-1, keepdims=True)
            e = jnp.exp(h - m)
            h = h - m - jnp.log(jnp.sum(e, axis=-1, keepdims=True))
        o_ref[...] = h

    return pl.pallas_call(
        body,
        grid=(_N // _TR,),
        in_specs=[
            pl.BlockSpec((_TR, _D), lambda i: (i, 0)),
            pl.BlockSpec((_NC, _TR, _D), lambda i: (0, i, 0)),
            pl.BlockSpec((_D, _D), lambda i: (0, 0)),
            pl.BlockSpec((1, _D), lambda i: (0, 0)),
        ],
        out_specs=pl.BlockSpec((_TR, _D), lambda i: (i, 0)),
        out_shape=jax.ShapeDtypeStruct((_N, _D), jnp.float32),
    )(x, p, w, b)


def kernel(features, edge_index, W1, b1, W2, b2):
    combo_r = (edge_index[0] + edge_index[1] * 65536).reshape(_NW, _K, _B)
    b1r = b1.reshape(1, _D)
    b2r = b2.reshape(1, _D)

    p1 = _sc_segment_sum(features, combo_r)
    h1 = _tc_linear(features, p1, W1, b1r, lsm=False)
    p2 = _sc_segment_sum(h1, combo_r)
    return _tc_linear(h1, p2, W2, b2r, lsm=True)


# speedup vs baseline: 14.3054x; 1.1954x over previous
import functools

import jax
import jax.numpy as jnp
from jax import lax
from jax.experimental import pallas as pl
from jax.experimental.pallas import tpu as pltpu
from jax.experimental.pallas import tpu_sc as plsc

_N = 10000
_E = 320000
_D = 128

_NC = 2
_NS = 16
_NW = _NC * _NS
_EPW = _E // _NW
_B = 80
_K = _EPW // _B
_WC = 25
_ROWS_PER_TILE = 640


def _sc_segment_sum(x, combo_r):
    mesh = plsc.VectorSubcoreMesh(core_axis_name="c", subcore_axis_name="s")

    @functools.partial(
        pl.kernel,
        out_type=jax.ShapeDtypeStruct((_NC, _N, _D), jnp.float32),
        mesh=mesh,
        scratch_types=[
            pltpu.VMEM((_WC, _B), jnp.int32),
            pltpu.VMEM((_WC, _B), jnp.int32),
            pltpu.VMEM((_B,), jnp.int32),
            pltpu.VMEM((_B,), jnp.int32),
            pltpu.VMEM((_B,), jnp.int32),
            pltpu.VMEM((_B,), jnp.int32),
            pltpu.VMEM((_B,), jnp.int32),
            pltpu.VMEM((_B,), jnp.int32),
            pltpu.VMEM((_B, _D), jnp.float32),
            pltpu.VMEM((_B, _D), jnp.float32),
            pltpu.VMEM((_B, _D), jnp.float32),
            pltpu.VMEM_SHARED((_N, _D), jnp.float32),
            pltpu.SemaphoreType.DMA,
            pltpu.SemaphoreType.DMA,
            pltpu.SemaphoreType.DMA,
            pltpu.SemaphoreType.DMA,
        ],
    )
    def k(x_hbm, combo_hbm, out_hbm, combo_w0, combo_w1,
          sidx0, sidx1, sidx2, didx0, didx1, didx2,
          rows0, rows1, rows2, acc_sh, sem0, sem1, sem2, semw):
        c = lax.axis_index("c")
        s = lax.axis_index("s")
        wid = s * _NC + c

        def zero_body(i, _):
            zr = i // (_D // 16)
            zc = (i % (_D // 16)) * 16
            rows0[zr, pl.ds(zc, 16)] = jnp.zeros((16,), jnp.float32)
            return 0

        lax.fori_loop(0, _B * (_D // 16), zero_body, 0)

        zbase = jnp.minimum(s * _ROWS_PER_TILE, _N - _ROWS_PER_TILE)
        for rblk in range(_ROWS_PER_TILE // _B):
            pltpu.sync_copy(rows0, acc_sh.at[pl.ds(zbase + rblk * _B, _B)])

        pltpu.sync_copy(combo_hbm.at[wid, 0], combo_w0)
        pltpu.async_copy(combo_hbm.at[wid, 1], combo_w1, semw)

        plsc.subcore_barrier()

        def _unp(cw, r, sidx, didx):
            for i in range(_B // 16):
                w = cw[r, pl.ds(i * 16, 16)]
                sidx[pl.ds(i * 16, 16)] = w & 0xFFFF
                didx[pl.ds(i * 16, 16)] = lax.shift_right_logical(w, 16)

        def unpack(x, sidx, didx):
            r = x % _WC

            @pl.when((x // _WC) % 2 == 0)
            def _():
                _unp(combo_w0, r, sidx, didx)

            @pl.when((x // _WC) % 2 == 1)
            def _():
                _unp(combo_w1, r, sidx, didx)

        def window_refill(x):
            @pl.when(x % _WC == 0)
            def _():
                pltpu.make_async_copy(
                    combo_hbm.at[wid, 0], combo_w0, semw
                ).wait()
                wnext = x // _WC + 1

                @pl.when(wnext < _K // _WC)
                def _():
                    @pl.when(wnext % 2 == 0)
                    def _():
                        pltpu.async_copy(combo_hbm.at[wid, wnext], combo_w0, semw)

                    @pl.when(wnext % 2 == 1)
                    def _():
                        pltpu.async_copy(combo_hbm.at[wid, wnext], combo_w1, semw)

        def wait_gather(sidx, buf, sem):
            pltpu.make_async_copy(x_hbm.at[sidx], buf, sem).wait()

        unpack(0, sidx0, didx0)
        pltpu.async_copy(x_hbm.at[sidx0], rows0, sem0)
        unpack(1, sidx1, didx1)
        pltpu.async_copy(x_hbm.at[sidx1], rows1, sem1)

        def step(cur, csem, cdidx, nxt, nsem, nsidx, ndidx, j):
            x = j + 2
            window_refill(x)
            unpack(x, nsidx, ndidx)
            pltpu.async_copy(x_hbm.at[nsidx], nxt, nsem)
            wait_gather(nsidx, cur, csem)
            pltpu.sync_copy(cur, acc_sh.at[cdidx], add=True)

        def body(j, _):
            @pl.when(j % 3 == 0)
            def _():
                step(rows0, sem0, didx0, rows2, sem2, sidx2, didx2, j)

            @pl.when(j % 3 == 1)
            def _():
                step(rows1, sem1, didx1, rows0, sem0, sidx0, didx0, j)

            @pl.when(j % 3 == 2)
            def _():
                step(rows2, sem2, didx2, rows1, sem1, sidx1, didx1, j)

            return 0

        lax.fori_loop(0, _K - 2, body, 0)
        wait_gather(sidx0, rows0, sem0)
        pltpu.sync_copy(rows0, acc_sh.at[didx0], add=True)
        wait_gather(sidx1, rows1, sem1)
        pltpu.sync_copy(rows1, acc_sh.at[didx1], add=True)

        plsc.subcore_barrier()

        @pl.when(s < _NS - 1)
        def _():
            pltpu.sync_copy(
                acc_sh.at[pl.ds(s * _ROWS_PER_TILE, _ROWS_PER_TILE)],
                out_hbm.at[c, pl.ds(s * _ROWS_PER_TILE, _ROWS_PER_TILE)],
            )

        @pl.when(s == _NS - 1)
        def _():
            last = (_NS - 1) * _ROWS_PER_TILE
            pltpu.sync_copy(
                acc_sh.at[pl.ds(last, _N - last)],
                out_hbm.at[c, pl.ds(last, _N - last)],
            )

    return k(x, combo_r)


_TR = 2000


def _tc_linear(x, p, w, b, lsm):
    def body(x_ref, p_ref, w_ref, b_ref, o_ref):
        rst = x_ref[...] + p_ref[0] + p_ref[1]
        h = jnp.dot(rst, w_ref[...], preferred_element_type=jnp.float32) + b_ref[...]
        if lsm:
            m = jnp.max(h, axis=-1, keepdims=True)
            e = jnp.exp(h - m)
            h = h - m - jnp.log(jnp.sum(e, axis=-1, keepdims=True))
        o_ref[...] = h

    return pl.pallas_call(
        body,
        grid=(_N // _TR,),
        in_specs=[
            pl.BlockSpec((_TR, _D), lambda i: (i, 0)),
            pl.BlockSpec((_NC, _TR, _D), lambda i: (0, i, 0)),
            pl.BlockSpec((_D, _D), lambda i: (0, 0)),
            pl.BlockSpec((1, _D), lambda i: (0, 0)),
        ],
        out_specs=pl.BlockSpec((_TR, _D), lambda i: (i, 0)),
        out_shape=jax.ShapeDtypeStruct((_N, _D), jnp.float32),
    )(x, p, w, b)


def kernel(features, edge_index, W1, b1, W2, b2):
    combo_r = (edge_index[0] + edge_index[1] * 65536).reshape(
        _NW, _K // _WC, _WC, _B
    )
    b1r = b1.reshape(1, _D)
    b2r = b2.reshape(1, _D)

    p1 = _sc_segment_sum(features, combo_r)
    h1 = _tc_linear(features, p1, W1, b1r, lsm=False)
    p2 = _sc_segment_sum(h1, combo_r)
    return _tc_linear(h1, p2, W2, b2r, lsm=True)


# speedup vs baseline: 14.3594x; 1.0038x over previous
import functools

import jax
import jax.numpy as jnp
from jax import lax
from jax.experimental import pallas as pl
from jax.experimental.pallas import tpu as pltpu
from jax.experimental.pallas import tpu_sc as plsc

_N = 10000
_E = 320000
_D = 128

_NC = 2
_NS = 16
_NW = _NC * _NS
_EPW = _E // _NW
_B = 80
_K = _EPW // _B
_WC = 25
_ROWS_PER_TILE = 640


def _sc_segment_sum(x, combo_r):
    mesh = plsc.VectorSubcoreMesh(core_axis_name="c", subcore_axis_name="s")

    @functools.partial(
        pl.kernel,
        out_type=jax.ShapeDtypeStruct((_NC, _N, _D), jnp.float32),
        mesh=mesh,
        scratch_types=[
            pltpu.VMEM((_WC, _B), jnp.int32),
            pltpu.VMEM((_WC, _B), jnp.int32),
            pltpu.VMEM((_B,), jnp.int32),
            pltpu.VMEM((_B,), jnp.int32),
            pltpu.VMEM((_B,), jnp.int32),
            pltpu.VMEM((_B,), jnp.int32),
            pltpu.VMEM((_B,), jnp.int32),
            pltpu.VMEM((_B,), jnp.int32),
            pltpu.VMEM((_B, _D), jnp.float32),
            pltpu.VMEM((_B, _D), jnp.float32),
            pltpu.VMEM((_B, _D), jnp.float32),
            pltpu.VMEM_SHARED((_N, _D), jnp.float32),
            pltpu.SemaphoreType.DMA,
            pltpu.SemaphoreType.DMA,
            pltpu.SemaphoreType.DMA,
            pltpu.SemaphoreType.DMA,
        ],
    )
    def k(x_hbm, combo_hbm, out_hbm, combo_w0, combo_w1,
          sidx0, sidx1, sidx2, didx0, didx1, didx2,
          rows0, rows1, rows2, acc_sh, sem0, sem1, sem2, semw):
        c = lax.axis_index("c")
        s = lax.axis_index("s")
        wid = s * _NC + c

        zbase = jnp.minimum(s * _ROWS_PER_TILE, _N - _ROWS_PER_TILE)

        @pl.when(c == 0)
        def _():
            pltpu.sync_copy(
                x_hbm.at[pl.ds(zbase, _ROWS_PER_TILE)],
                acc_sh.at[pl.ds(zbase, _ROWS_PER_TILE)],
            )

        @pl.when(c == 1)
        def _():
            def zero_body(i, _):
                zr = i // (_D // 16)
                zc = (i % (_D // 16)) * 16
                rows0[zr, pl.ds(zc, 16)] = jnp.zeros((16,), jnp.float32)
                return 0

            lax.fori_loop(0, _B * (_D // 16), zero_body, 0)
            for rblk in range(_ROWS_PER_TILE // _B):
                pltpu.sync_copy(rows0, acc_sh.at[pl.ds(zbase + rblk * _B, _B)])

        pltpu.sync_copy(combo_hbm.at[wid, 0], combo_w0)
        pltpu.async_copy(combo_hbm.at[wid, 1], combo_w1, semw)

        plsc.subcore_barrier()

        def _unp(cw, r, sidx, didx):
            for i in range(_B // 16):
                w = cw[r, pl.ds(i * 16, 16)]
                sidx[pl.ds(i * 16, 16)] = w & 0xFFFF
                didx[pl.ds(i * 16, 16)] = lax.shift_right_logical(w, 16)

        def unpack(x, sidx, didx):
            r = x % _WC

            @pl.when((x // _WC) % 2 == 0)
            def _():
                _unp(combo_w0, r, sidx, didx)

            @pl.when((x // _WC) % 2 == 1)
            def _():
                _unp(combo_w1, r, sidx, didx)

        def window_refill(x):
            @pl.when(x % _WC == 0)
            def _():
                pltpu.make_async_copy(
                    combo_hbm.at[wid, 0], combo_w0, semw
                ).wait()
                wnext = x // _WC + 1

                @pl.when(wnext < _K // _WC)
                def _():
                    @pl.when(wnext % 2 == 0)
                    def _():
                        pltpu.async_copy(combo_hbm.at[wid, wnext], combo_w0, semw)

                    @pl.when(wnext % 2 == 1)
                    def _():
                        pltpu.async_copy(combo_hbm.at[wid, wnext], combo_w1, semw)

        def wait_gather(sidx, buf, sem):
            pltpu.make_async_copy(x_hbm.at[sidx], buf, sem).wait()

        unpack(0, sidx0, didx0)
        pltpu.async_copy(x_hbm.at[sidx0], rows0, sem0)
        unpack(1, sidx1, didx1)
        pltpu.async_copy(x_hbm.at[sidx1], rows1, sem1)

        def step(cur, csem, cdidx, nxt, nsem, nsidx, ndidx, j):
            x = j + 2
            window_refill(x)
            unpack(x, nsidx, ndidx)
            pltpu.async_copy(x_hbm.at[nsidx], nxt, nsem)
            wait_gather(nsidx, cur, csem)
            pltpu.sync_copy(cur, acc_sh.at[cdidx], add=True)

        def body(j, _):
            @pl.when(j % 3 == 0)
            def _():
                step(rows0, sem0, didx0, rows2, sem2, sidx2, didx2, j)

            @pl.when(j % 3 == 1)
            def _():
                step(rows1, sem1, didx1, rows0, sem0, sidx0, didx0, j)

            @pl.when(j % 3 == 2)
            def _():
                step(rows2, sem2, didx2, rows1, sem1, sidx1, didx1, j)

            return 0

        lax.fori_loop(0, _K - 2, body, 0)
        wait_gather(sidx0, rows0, sem0)
        pltpu.sync_copy(rows0, acc_sh.at[didx0], add=True)
        wait_gather(sidx1, rows1, sem1)
        pltpu.sync_copy(rows1, acc_sh.at[didx1], add=True)

        plsc.subcore_barrier()

        @pl.when(s < _NS - 1)
        def _():
            pltpu.sync_copy(
                acc_sh.at[pl.ds(s * _ROWS_PER_TILE, _ROWS_PER_TILE)],
                out_hbm.at[c, pl.ds(s * _ROWS_PER_TILE, _ROWS_PER_TILE)],
            )

        @pl.when(s == _NS - 1)
        def _():
            last = (_NS - 1) * _ROWS_PER_TILE
            pltpu.sync_copy(
                acc_sh.at[pl.ds(last, _N - last)],
                out_hbm.at[c, pl.ds(last, _N - last)],
            )

    return k(x, combo_r)


_TR = 2000


def _tc_linear(p, w, b, lsm):
    def body(p_ref, w_ref, b_ref, o_ref):
        rst = p_ref[0] + p_ref[1]
        h = jnp.dot(rst, w_ref[...], preferred_element_type=jnp.float32) + b_ref[...]
        if lsm:
            m = jnp.max(h, axis=-1, keepdims=True)
            e = jnp.exp(h - m)
            h = h - m - jnp.log(jnp.sum(e, axis=-1, keepdims=True))
        o_ref[...] = h

    return pl.pallas_call(
        body,
        grid=(_N // _TR,),
        in_specs=[
            pl.BlockSpec((_NC, _TR, _D), lambda i: (0, i, 0)),
            pl.BlockSpec((_D, _D), lambda i: (0, 0)),
            pl.BlockSpec((1, _D), lambda i: (0, 0)),
        ],
        out_specs=pl.BlockSpec((_TR, _D), lambda i: (i, 0)),
        out_shape=jax.ShapeDtypeStruct((_N, _D), jnp.float32),
    )(p, w, b)


def kernel(features, edge_index, W1, b1, W2, b2):
    combo_r = (edge_index[0] + edge_index[1] * 65536).reshape(
        _NW, _K // _WC, _WC, _B
    )
    b1r = b1.reshape(1, _D)
    b2r = b2.reshape(1, _D)

    p1 = _sc_segment_sum(features, combo_r)
    h1 = _tc_linear(p1, W1, b1r, lsm=False)
    p2 = _sc_segment_sum(h1, combo_r)
    return _tc_linear(p2, W2, b2r, lsm=True)
